# baseline (speedup 1.0000x reference)
import jax
import jax.numpy as jnp
import numpy as np
from jax.experimental import pallas as pl
from jax.experimental.pallas import tpu as pltpu

BLOCK_ROWS = 10000


def _forest_kernel(x1_ref, x2_ref, w_ref, thr_ref, cr_ref, kpat_ref, vflat_ref,
                   o1_ref, o2_ref):
    for x_ref, o_ref in ((x1_ref, o1_ref), (x2_ref, o2_ref)):
        proj = jnp.dot(x_ref[:], w_ref[:], preferred_element_type=jnp.float32)
        bits = (proj > thr_ref[:]).astype(jnp.float32)
        rep = jnp.dot(bits, cr_ref[:], preferred_element_type=jnp.float32)
        picked = jnp.where(rep == kpat_ref[:], vflat_ref[:], 0.0)
        o_ref[:] = jnp.sum(picked, axis=1, keepdims=True)


def kernel(x, splits, thresholds, values):
    n, d = x.shape
    t, depth = thresholds.shape
    leaves = values.shape[1]
    w = splits.reshape(t * depth, d).T
    thr = thresholds.reshape(1, t * depth)
    cr = np.zeros((t * depth, t * leaves), dtype=np.float32)
    for ti in range(t):
        for dd in range(depth):
            cr[ti * depth + dd, ti * leaves:(ti + 1) * leaves] = 2.0 ** (depth - 1 - dd)
    cr = jnp.asarray(cr)
    kpat = jnp.asarray(np.tile(np.arange(leaves, dtype=np.float32), t)[None, :])
    vflat = (jnp.roll(values, 1, axis=1) / t).reshape(1, t * leaves)

    half_blocks = n // (2 * BLOCK_ROWS)
    grid = (half_blocks,)
    o1, o2 = pl.pallas_call(
        _forest_kernel,
        grid=grid,
        in_specs=[
            pl.BlockSpec((BLOCK_ROWS, d), lambda i: (i, 0)),
            pl.BlockSpec((BLOCK_ROWS, d), lambda i: (i + n // (2 * BLOCK_ROWS), 0)),
            pl.BlockSpec((d, t * depth), lambda i: (0, 0)),
            pl.BlockSpec((1, t * depth), lambda i: (0, 0)),
            pl.BlockSpec((t * depth, t * leaves), lambda i: (0, 0)),
            pl.BlockSpec((1, t * leaves), lambda i: (0, 0)),
            pl.BlockSpec((1, t * leaves), lambda i: (0, 0)),
        ],
        out_specs=[
            pl.BlockSpec((BLOCK_ROWS, 1), lambda i: (i, 0)),
            pl.BlockSpec((BLOCK_ROWS, 1), lambda i: (i, 0)),
        ],
        out_shape=[
            jax.ShapeDtypeStruct((n // 2, 1), jnp.float32),
            jax.ShapeDtypeStruct((n // 2, 1), jnp.float32),
        ],
        compiler_params=pltpu.CompilerParams(
            dimension_semantics=("parallel",)),
    )(x, x, w, thr, cr, kpat, vflat)
    return jnp.concatenate([o1, o2], axis=0)


# speedup vs baseline: 1.0032x; 1.0032x over previous
import jax
import jax.numpy as jnp
import numpy as np
from jax.experimental import pallas as pl
from jax.experimental.pallas import tpu as pltpu

BLOCK_ROWS = 20000


def _forest_kernel(x_ref, w_ref, thr_ref, cr_ref, kpat_ref, vflat_ref, out_ref):
    proj = jnp.dot(x_ref[:], w_ref[:], preferred_element_type=jnp.float32)
    bits = (proj > thr_ref[:]).astype(jnp.float32)
    rep = jnp.dot(bits, cr_ref[:], preferred_element_type=jnp.float32)
    picked = jnp.where(rep == kpat_ref[:], vflat_ref[:], 0.0)
    out_ref[:] = jnp.sum(picked, axis=1, keepdims=True)


def kernel(x, splits, thresholds, values):
    n, d = x.shape
    t, depth = thresholds.shape
    leaves = values.shape[1]
    w = splits.reshape(t * depth, d).T
    thr = thresholds.reshape(1, t * depth)
    cr = np.zeros((t * depth, t * leaves), dtype=np.float32)
    for ti in range(t):
        for dd in range(depth):
            cr[ti * depth + dd, ti * leaves:(ti + 1) * leaves] = 2.0 ** (depth - 1 - dd)
    cr = jnp.asarray(cr)
    kpat = jnp.asarray(np.tile(np.arange(leaves, dtype=np.float32), t)[None, :])
    vflat = (jnp.roll(values, 1, axis=1) / t).reshape(1, t * leaves)

    grid = (n // BLOCK_ROWS,)
    return pl.pallas_call(
        _forest_kernel,
        grid=grid,
        in_specs=[
            pl.BlockSpec((BLOCK_ROWS, d), lambda i: (i, 0)),
            pl.BlockSpec((d, t * depth), lambda i: (0, 0)),
            pl.BlockSpec((1, t * depth), lambda i: (0, 0)),
            pl.BlockSpec((t * depth, t * leaves), lambda i: (0, 0)),
            pl.BlockSpec((1, t * leaves), lambda i: (0, 0)),
            pl.BlockSpec((1, t * leaves), lambda i: (0, 0)),
        ],
        out_specs=pl.BlockSpec((BLOCK_ROWS, 1), lambda i: (i, 0)),
        out_shape=jax.ShapeDtypeStruct((n, 1), jnp.float32),
        compiler_params=pltpu.CompilerParams(
            dimension_semantics=("parallel",),
            vmem_limit_bytes=110 * 1024 * 1024),
    )(x, w, thr, cr, kpat, vflat)
